# Initial kernel scaffold; baseline (speedup 1.0000x reference)
import functools

import jax
import jax.numpy as jnp
from jax import lax
from jax.experimental import pallas as pl
from jax.experimental.pallas import tpu as pltpu
from jax.experimental.pallas import tpu_sc as plsc

B = 4096
L = 200
D = 64
NC = 2
NS = 16
NW = NC * NS
RPW = B // NW
LP = 208
SPLIT = 128


@functools.partial(
    pl.kernel,
    out_type=jax.ShapeDtypeStruct((B, D), jnp.float32),
    mesh=plsc.VectorSubcoreMesh(core_axis_name="c", subcore_axis_name="s"),
    scratch_types=[
        pltpu.VMEM((RPW, LP), jnp.int32),
        pltpu.VMEM((L, D), jnp.float32),
        pltpu.VMEM((L, D), jnp.float32),
        pltpu.VMEM((RPW, D), jnp.float32),
        pltpu.SemaphoreType.DMA,
        pltpu.SemaphoreType.DMA,
    ],
)
def _avg_embed(code_h, table_h, out_h, idx_v, buf0, buf1, out_v, sem0, sem1):
    wid = lax.axis_index("s") * NC + lax.axis_index("c")
    base = wid * RPW

    pltpu.sync_copy(code_h.at[pl.ds(base, RPW)], idx_v.at[:, pl.ds(0, L)])

    bufs = (buf0, buf1)
    sems = (sem0, sem1)

    def start(r, b):
        pltpu.async_copy(
            table_h.at[idx_v.at[r, pl.ds(0, SPLIT)]],
            bufs[b].at[pl.ds(0, SPLIT)],
            sems[b],
        )
        pltpu.async_copy(
            table_h.at[idx_v.at[r, pl.ds(SPLIT, L - SPLIT)]],
            bufs[b].at[pl.ds(SPLIT, L - SPLIT)],
            sems[b],
        )

    def wait(b):
        pltpu.make_async_copy(table_h.at[pl.ds(0, L)], bufs[b], sems[b]).wait()

    lane = lax.iota(jnp.int32, 16)

    def reduce_row(buf, r):
        def cbody(k, cv):
            v = idx_v[r, pl.ds(k * 16, 16)]
            return cv + jnp.where(v != 0, 1.0, 0.0).astype(jnp.float32)

        cv = lax.fori_loop(0, 12, cbody, jnp.zeros((16,), jnp.float32),
                           unroll=4)
        vtail = idx_v[r, pl.ds(192, 16)]
        cv = cv + jnp.where((vtail != 0) & (lane < 8), 1.0, 0.0).astype(
            jnp.float32)
        cnt = jnp.broadcast_to(jnp.sum(cv), (16,))

        def sbody(l, accs):
            a0, a1, a2, a3 = accs
            return (
                a0 + buf[l, pl.ds(0, 16)],
                a1 + buf[l, pl.ds(16, 16)],
                a2 + buf[l, pl.ds(32, 16)],
                a3 + buf[l, pl.ds(48, 16)],
            )

        z = jnp.zeros((16,), jnp.float32)
        a0, a1, a2, a3 = lax.fori_loop(0, L, sbody, (z, z, z, z), unroll=8)
        out_v[r, pl.ds(0, 16)] = a0 / cnt
        out_v[r, pl.ds(16, 16)] = a1 / cnt
        out_v[r, pl.ds(32, 16)] = a2 / cnt
        out_v[r, pl.ds(48, 16)] = a3 / cnt

    start(0, 0)

    def gbody(g, carry):
        r0 = 2 * g
        start(r0 + 1, 1)
        wait(0)
        reduce_row(buf0, r0)

        @pl.when(g < RPW // 2 - 1)
        def _():
            start(r0 + 2, 0)

        wait(1)
        reduce_row(buf1, r0 + 1)
        return carry

    lax.fori_loop(0, RPW // 2, gbody, 0)

    pltpu.sync_copy(out_v, out_h.at[pl.ds(base, RPW)])


def kernel(code, code_table):
    return _avg_embed(code.astype(jnp.int32), code_table)


# speedup vs baseline: 16.9097x; 16.9097x over previous
import functools

import jax
import jax.numpy as jnp
from jax import lax
from jax.experimental import pallas as pl
from jax.experimental.pallas import tpu as pltpu
from jax.experimental.pallas import tpu_sc as plsc

B = 4096
L = 200
D = 64
NC = 2
NS = 16
NW = NC * NS
RPW = B // NW
LP = 208
SPLIT = 128


@functools.partial(
    pl.kernel,
    out_type=jax.ShapeDtypeStruct((B, D), jnp.float32),
    mesh=plsc.VectorSubcoreMesh(core_axis_name="c", subcore_axis_name="s"),
    compiler_params=pltpu.CompilerParams(
        use_tc_tiling_on_sc=False, needs_layout_passes=False),
    scratch_types=[
        pltpu.VMEM((RPW, LP), jnp.int32),
        pltpu.VMEM((L, D), jnp.float32),
        pltpu.VMEM((L, D), jnp.float32),
        pltpu.VMEM((RPW, D), jnp.float32),
        pltpu.SemaphoreType.DMA,
        pltpu.SemaphoreType.DMA,
    ],
)
def _avg_embed(code_h, table_h, out_h, idx_v, buf0, buf1, out_v, sem0, sem1):
    wid = lax.axis_index("s") * NC + lax.axis_index("c")
    base = wid * RPW

    pltpu.sync_copy(code_h.at[pl.ds(base, RPW)], idx_v.at[:, pl.ds(0, L)])

    bufs = (buf0, buf1)
    sems = (sem0, sem1)

    def start(r, b):
        pltpu.async_copy(
            table_h.at[idx_v.at[r, pl.ds(0, SPLIT)]],
            bufs[b].at[pl.ds(0, SPLIT)],
            sems[b],
        )
        pltpu.async_copy(
            table_h.at[idx_v.at[r, pl.ds(SPLIT, L - SPLIT)]],
            bufs[b].at[pl.ds(SPLIT, L - SPLIT)],
            sems[b],
        )

    def wait(b):
        pltpu.make_async_copy(table_h.at[pl.ds(0, L)], bufs[b], sems[b]).wait()

    lane = lax.iota(jnp.int32, 16)

    def reduce_row(buf, r):
        def cbody(k, cv):
            v = idx_v[r, pl.ds(k * 16, 16)]
            return cv + jnp.where(v != 0, 1.0, 0.0).astype(jnp.float32)

        cv = lax.fori_loop(0, 12, cbody, jnp.zeros((16,), jnp.float32),
                           unroll=4)
        vtail = idx_v[r, pl.ds(192, 16)]
        cv = cv + jnp.where((vtail != 0) & (lane < 8), 1.0, 0.0).astype(
            jnp.float32)
        cnt = jnp.broadcast_to(jnp.sum(cv), (16,))

        def sbody(l, accs):
            a0, a1, a2, a3 = accs
            return (
                a0 + buf[l, pl.ds(0, 16)],
                a1 + buf[l, pl.ds(16, 16)],
                a2 + buf[l, pl.ds(32, 16)],
                a3 + buf[l, pl.ds(48, 16)],
            )

        z = jnp.zeros((16,), jnp.float32)
        a0, a1, a2, a3 = lax.fori_loop(0, L, sbody, (z, z, z, z), unroll=8)
        out_v[r, pl.ds(0, 16)] = a0 / cnt
        out_v[r, pl.ds(16, 16)] = a1 / cnt
        out_v[r, pl.ds(32, 16)] = a2 / cnt
        out_v[r, pl.ds(48, 16)] = a3 / cnt

    start(0, 0)

    def gbody(g, carry):
        r0 = 2 * g
        start(r0 + 1, 1)
        wait(0)
        reduce_row(buf0, r0)

        @pl.when(g < RPW // 2 - 1)
        def _():
            start(r0 + 2, 0)

        wait(1)
        reduce_row(buf1, r0 + 1)
        return carry

    lax.fori_loop(0, RPW // 2, gbody, 0)

    pltpu.sync_copy(out_v, out_h.at[pl.ds(base, RPW)])


def kernel(code, code_table):
    return _avg_embed(code.astype(jnp.int32), code_table)
